# baseline (device time: 42515 ns/iter reference)
import jax
import jax.numpy as jnp
from jax import lax
from jax.experimental import pallas as pl
from jax.experimental.pallas import tpu as pltpu

M, N = 512, 512
N_STAGES = 5
C = 8
ROWS = M // C


def kernel(x):
    x2 = x.reshape(M, N)

    def body(x_ref, out_ref, acc_ref, send_bufs, recv_bufs, send_sems, recv_sems):
        my_x = lax.axis_index("x")
        my_y = lax.axis_index("y")
        my_z = lax.axis_index("z")

        partners = [
            (1 - my_x, my_y, my_z),
            (my_x, my_y ^ 1, my_z),
            (my_x, my_y ^ 2, my_z),
            (my_x, my_y, my_z ^ 1),
            (my_x, my_y, my_z ^ 2),
        ]

        barrier_sem = pltpu.get_barrier_semaphore()
        for p in partners:
            pl.semaphore_signal(
                barrier_sem, inc=1, device_id=p,
                device_id_type=pl.DeviceIdType.MESH,
            )
        pl.semaphore_wait(barrier_sem, N_STAGES)

        acc_ref[...] = x_ref[...].astype(jnp.bfloat16)

        rdmas = {}
        for s in range(N_STAGES):
            for c in range(C):
                rows = pl.ds(c * ROWS, ROWS)
                if s > 0:
                    rdmas[(s - 1, c)].wait_recv()
                    acc_ref[rows, :] += recv_bufs[s - 1, rows, :]
                send_bufs[s, rows, :] = acc_ref[rows, :]
                r = pltpu.make_async_remote_copy(
                    src_ref=send_bufs.at[s, rows, :],
                    dst_ref=recv_bufs.at[s, rows, :],
                    send_sem=send_sems.at[s, c],
                    recv_sem=recv_sems.at[s, c],
                    device_id=partners[s],
                    device_id_type=pl.DeviceIdType.MESH,
                )
                r.start()
                rdmas[(s, c)] = r

        s = N_STAGES - 1
        for c in range(C):
            rows = pl.ds(c * ROWS, ROWS)
            rdmas[(s, c)].wait_recv()
            acc_ref[rows, :] += recv_bufs[s, rows, :]
        out_ref[...] = acc_ref[...].astype(jnp.float32)

        for s in range(N_STAGES):
            for c in range(C):
                rdmas[(s, c)].wait_send()

    return pl.pallas_call(
        body,
        out_shape=jax.ShapeDtypeStruct((M, N), jnp.float32),
        in_specs=[pl.BlockSpec(memory_space=pltpu.VMEM)],
        out_specs=pl.BlockSpec(memory_space=pltpu.VMEM),
        scratch_shapes=[
            pltpu.VMEM((M, N), jnp.bfloat16),
            pltpu.VMEM((N_STAGES, M, N), jnp.bfloat16),
            pltpu.VMEM((N_STAGES, M, N), jnp.bfloat16),
            pltpu.SemaphoreType.DMA((N_STAGES, C)),
            pltpu.SemaphoreType.DMA((N_STAGES, C)),
        ],
        compiler_params=pltpu.CompilerParams(collective_id=0),
    )(x2)


# device time: 36050 ns/iter; 1.1793x vs baseline; 1.1793x over previous
import jax
import jax.numpy as jnp
from jax import lax
from jax.experimental import pallas as pl
from jax.experimental.pallas import tpu as pltpu

M, N = 512, 512


def kernel(x):
    x2 = x.reshape(M, N)

    def body(x_ref, out_ref, acc_ref, recv_buf, send_sems, recv_sems):
        my_x = lax.axis_index("x")
        my_y = lax.axis_index("y")
        my_z = lax.axis_index("z")

        b0z = my_z & 1
        b1z = my_z >> 1
        b0y = my_y & 1
        b1y = my_y >> 1

        p_z1 = (my_x, my_y, my_z ^ 1)
        p_z2 = (my_x, my_y, my_z ^ 2)
        p_y1 = (my_x, my_y ^ 1, my_z)
        p_y2 = (my_x, my_y ^ 2, my_z)
        p_x = (1 - my_x, my_y, my_z)

        base_z = b0z * 256 + b1z * 128
        keep64 = base_z + b0y * 64
        keep32 = keep64 + b1y * 32

        stages = [
            (p_z1, (1 - b0z) * 256, 256, 0, b0z * 256),
            (p_z2, b0z * 256 + (1 - b1z) * 128, 128, 256, base_z),
            (p_y1, base_z + (1 - b0y) * 64, 64, 384, keep64),
            (p_y2, keep64 + (1 - b1y) * 32, 32, 448, keep32),
            (p_x, keep32, 32, 480, keep32),
            (p_y2, keep32, 32, None, None),
            (p_y1, keep64, 64, None, None),
            (p_z2, base_z, 128, None, None),
            (p_z1, b0z * 256, 256, None, None),
        ]

        barrier_sem = pltpu.get_barrier_semaphore()
        for p in (p_z1, p_z2, p_y1, p_y2, p_x):
            pl.semaphore_signal(
                barrier_sem, inc=1, device_id=p,
                device_id_type=pl.DeviceIdType.MESH,
            )
        pl.semaphore_wait(barrier_sem, 5)

        acc_ref[...] = x_ref[...].astype(jnp.bfloat16)

        for s, (p, soff, nrows, roff, aoff) in enumerate(stages):
            src = acc_ref.at[pl.ds(soff, nrows), :]
            if roff is None:
                dst = acc_ref.at[pl.ds(soff, nrows), :]
            else:
                dst = recv_buf.at[pl.ds(roff, nrows), :]
            rdma = pltpu.make_async_remote_copy(
                src_ref=src,
                dst_ref=dst,
                send_sem=send_sems.at[s],
                recv_sem=recv_sems.at[s],
                device_id=p,
                device_id_type=pl.DeviceIdType.MESH,
            )
            rdma.start()
            rdma.wait()
            if roff is not None:
                acc_ref[pl.ds(aoff, nrows), :] += recv_buf[
                    pl.ds(roff, nrows), :
                ]

        out_ref[...] = acc_ref[...].astype(jnp.float32)

    return pl.pallas_call(
        body,
        out_shape=jax.ShapeDtypeStruct((M, N), jnp.float32),
        in_specs=[pl.BlockSpec(memory_space=pltpu.VMEM)],
        out_specs=pl.BlockSpec(memory_space=pltpu.VMEM),
        scratch_shapes=[
            pltpu.VMEM((M, N), jnp.bfloat16),
            pltpu.VMEM((M, N), jnp.bfloat16),
            pltpu.SemaphoreType.DMA((9,)),
            pltpu.SemaphoreType.DMA((9,)),
        ],
        compiler_params=pltpu.CompilerParams(collective_id=0),
    )(x2)


# device time: 30820 ns/iter; 1.3795x vs baseline; 1.1697x over previous
import jax
import jax.numpy as jnp
from jax import lax
from jax.experimental import pallas as pl
from jax.experimental.pallas import tpu as pltpu

M, N = 512, 512
N_STAGES = 9


def kernel(x):
    x2 = x.reshape(M, N)

    def body(x_ref, out_ref, acc_ref, recv_buf, send_sems, recv_sems):
        my_x = lax.axis_index("x")
        my_y = lax.axis_index("y")
        my_z = lax.axis_index("z")

        b0z, b1z = my_z & 1, my_z >> 1
        b0y, b1y = my_y & 1, my_y >> 1

        p_z1 = (my_x, my_y, my_z ^ 1)
        p_z2 = (my_x, my_y, my_z ^ 2)
        p_y1 = (my_x, my_y ^ 1, my_z)
        p_y2 = (my_x, my_y ^ 2, my_z)
        p_x = (1 - my_x, my_y, my_z)

        def half_stages(base, pl1_1, pl1_2, pl2_1, pl2_2, b0_1, b1_1, b0_2, b1_2):
            owned64 = base + b0_1 * 128 + b1_1 * 64
            keep32 = owned64 + b0_2 * 32
            keep16 = keep32 + b1_2 * 16
            return [
                (pl1_1, base + (1 - b0_1) * 128, 128, base + 0, base + b0_1 * 128),
                (pl1_2, base + b0_1 * 128 + (1 - b1_1) * 64, 64, base + 128, owned64),
                (pl2_1, owned64 + (1 - b0_2) * 32, 32, base + 192, keep32),
                (pl2_2, keep32 + (1 - b1_2) * 16, 16, base + 224, keep16),
                (p_x, keep16, 16, base + 240, keep16),
                (pl2_2, keep16, 16, None, None),
                (pl2_1, keep32, 32, None, None),
                (pl1_2, owned64, 64, None, None),
                (pl1_1, base + b0_1 * 128, 128, None, None),
            ]

        halves = [
            half_stages(0, p_z1, p_z2, p_y1, p_y2, b0z, b1z, b0y, b1y),
            half_stages(256, p_y1, p_y2, p_z1, p_z2, b0y, b1y, b0z, b1z),
        ]

        barrier_sem = pltpu.get_barrier_semaphore()
        for p in (p_z1, p_z2, p_y1, p_y2, p_x):
            pl.semaphore_signal(
                barrier_sem, inc=1, device_id=p,
                device_id_type=pl.DeviceIdType.MESH,
            )
        pl.semaphore_wait(barrier_sem, 5)

        acc_ref[...] = x_ref[...].astype(jnp.bfloat16)

        rdmas = {}

        def start(h, k):
            p, soff, nrows, roff, _ = halves[h][k]
            src = acc_ref.at[pl.ds(soff, nrows), :]
            if roff is None:
                dst = acc_ref.at[pl.ds(soff, nrows), :]
            else:
                dst = recv_buf.at[pl.ds(roff, nrows), :]
            r = pltpu.make_async_remote_copy(
                src_ref=src,
                dst_ref=dst,
                send_sem=send_sems.at[h, k],
                recv_sem=recv_sems.at[h, k],
                device_id=p,
                device_id_type=pl.DeviceIdType.MESH,
            )
            r.start()
            rdmas[(h, k)] = r

        start(0, 0)
        start(1, 0)
        for k in range(N_STAGES):
            for h in (0, 1):
                _, _, nrows, roff, aoff = halves[h][k]
                rdmas[(h, k)].wait_recv()
                if roff is not None:
                    acc_ref[pl.ds(aoff, nrows), :] += recv_buf[
                        pl.ds(roff, nrows), :
                    ]
                if k + 1 < N_STAGES:
                    start(h, k + 1)

        out_ref[...] = acc_ref[...].astype(jnp.float32)

        for h in (0, 1):
            for k in range(N_STAGES):
                rdmas[(h, k)].wait_send()

    return pl.pallas_call(
        body,
        out_shape=jax.ShapeDtypeStruct((M, N), jnp.float32),
        in_specs=[pl.BlockSpec(memory_space=pltpu.VMEM)],
        out_specs=pl.BlockSpec(memory_space=pltpu.VMEM),
        scratch_shapes=[
            pltpu.VMEM((M, N), jnp.bfloat16),
            pltpu.VMEM((M, N), jnp.bfloat16),
            pltpu.SemaphoreType.DMA((2, N_STAGES)),
            pltpu.SemaphoreType.DMA((2, N_STAGES)),
        ],
        compiler_params=pltpu.CompilerParams(collective_id=0),
    )(x2)


# device time: 23994 ns/iter; 1.7719x vs baseline; 1.2845x over previous
import jax
import jax.numpy as jnp
from jax import lax
from jax.experimental import pallas as pl
from jax.experimental.pallas import tpu as pltpu

M, N = 512, 512
ROWS = 64


def kernel(x):
    x2 = x.reshape(M, N)

    def body(x_ref, out_ref, acc_ref, recv_buf, send_sems, recv_sems):
        my_x = lax.axis_index("x")
        my_y = lax.axis_index("y")
        my_z = lax.axis_index("z")

        def peer_l1(h, c):
            return (my_x, my_y, c) if h == 0 else (my_x, c, my_z)

        def peer_l2(h, c):
            return (my_x, c, my_z) if h == 0 else (my_x, my_y, c)

        l1 = [my_z, my_y]
        l2 = [my_y, my_z]
        base = [0, 256]
        p_x = (1 - my_x, my_y, my_z)

        barrier_sem = pltpu.get_barrier_semaphore()
        peers = [(my_x, my_y, (my_z + d) % 4) for d in (1, 2, 3)]
        peers += [(my_x, (my_y + d) % 4, my_z) for d in (1, 2, 3)]
        peers += [p_x]
        for p in peers:
            pl.semaphore_signal(
                barrier_sem, inc=1, device_id=p,
                device_id_type=pl.DeviceIdType.MESH,
            )
        pl.semaphore_wait(barrier_sem, 7)

        acc_ref[...] = x_ref[...].astype(jnp.bfloat16)

        rdmas = {}

        def a2a_start(h, stage, src_off_fn, dst_acc):
            coord = l1[h] if stage in (0, 3) else l2[h]
            for d in (1, 2, 3):
                pc = (coord + d) % 4
                p = peer_l1(h, pc) if stage in (0, 3) else peer_l2(h, pc)
                soff = src_off_fn(pc)
                src = acc_ref.at[pl.ds(soff, ROWS), :]
                if dst_acc:
                    dst = acc_ref.at[pl.ds(soff, ROWS), :]
                else:
                    dst = recv_buf.at[h, 3 * (stage == 1) + (d - 1)]
                r = pltpu.make_async_remote_copy(
                    src_ref=src,
                    dst_ref=dst,
                    send_sem=send_sems.at[h, stage, d - 1],
                    recv_sem=recv_sems.at[h, stage, d - 1],
                    device_id=p,
                    device_id_type=pl.DeviceIdType.MESH,
                )
                r.start()
                rdmas[(h, stage, d)] = r

        def own_off(h):
            return base[h] + l1[h] * ROWS

        for h in (0, 1):
            a2a_start(h, 0, lambda pc, h=h: base[h] + pc * ROWS, False)
        for h in (0, 1):
            for d in (1, 2, 3):
                rdmas[(h, 0, d)].wait_recv()
            acc_ref[pl.ds(own_off(h), ROWS), :] += (
                recv_buf[h, 0] + recv_buf[h, 1] + recv_buf[h, 2]
            )
            a2a_start(h, 1, lambda pc, h=h: own_off(h), False)
        for h in (0, 1):
            for d in (1, 2, 3):
                rdmas[(h, 1, d)].wait_recv()
                rdmas[(h, 1, d)].wait_send()
            acc_ref[pl.ds(own_off(h), ROWS), :] += (
                recv_buf[h, 3] + recv_buf[h, 4] + recv_buf[h, 5]
            )
            r = pltpu.make_async_remote_copy(
                src_ref=acc_ref.at[pl.ds(own_off(h), ROWS), :],
                dst_ref=recv_buf.at[h, 6],
                send_sem=send_sems.at[h, 2, 0],
                recv_sem=recv_sems.at[h, 2, 0],
                device_id=p_x,
                device_id_type=pl.DeviceIdType.MESH,
            )
            r.start()
            rdmas[(h, 2, 1)] = r
        for h in (0, 1):
            rdmas[(h, 2, 1)].wait_recv()
            rdmas[(h, 2, 1)].wait_send()
            acc_ref[pl.ds(own_off(h), ROWS), :] += recv_buf[h, 6]
            a2a_start(h, 3, lambda pc, h=h: own_off(h), True)
        for h in (0, 1):
            for d in (1, 2, 3):
                rdmas[(h, 3, d)].wait_recv()

        out_ref[...] = acc_ref[...].astype(jnp.float32)

        for h in (0, 1):
            for d in (1, 2, 3):
                rdmas[(h, 0, d)].wait_send()
                rdmas[(h, 3, d)].wait_send()

    return pl.pallas_call(
        body,
        out_shape=jax.ShapeDtypeStruct((M, N), jnp.float32),
        in_specs=[pl.BlockSpec(memory_space=pltpu.VMEM)],
        out_specs=pl.BlockSpec(memory_space=pltpu.VMEM),
        scratch_shapes=[
            pltpu.VMEM((M, N), jnp.bfloat16),
            pltpu.VMEM((2, 7, ROWS, N), jnp.bfloat16),
            pltpu.SemaphoreType.DMA((2, 4, 3)),
            pltpu.SemaphoreType.DMA((2, 4, 3)),
        ],
        compiler_params=pltpu.CompilerParams(collective_id=0),
    )(x2)


# device time: 23224 ns/iter; 1.8306x vs baseline; 1.0332x over previous
import jax
import jax.numpy as jnp
from jax import lax
from jax.experimental import pallas as pl
from jax.experimental.pallas import tpu as pltpu

M, N = 512, 512
ROWS = 64
SUBS = 2
SR = ROWS // SUBS


def kernel(x):
    x2 = x.reshape(M, N)

    def body(x_ref, out_ref, acc_ref, recv_buf, send_sems, recv_sems):
        my_x = lax.axis_index("x")
        my_y = lax.axis_index("y")
        my_z = lax.axis_index("z")

        l1 = [my_z, my_y]
        l2 = [my_y, my_z]
        base = [0, 256]
        p_x = (1 - my_x, my_y, my_z)

        def peer_l1(h, c):
            return (my_x, my_y, c) if h == 0 else (my_x, c, my_z)

        def peer_l2(h, c):
            return (my_x, c, my_z) if h == 0 else (my_x, my_y, c)

        barrier_sem = pltpu.get_barrier_semaphore()
        peers = [(my_x, my_y, (my_z + d) % 4) for d in (1, 2, 3)]
        peers += [(my_x, (my_y + d) % 4, my_z) for d in (1, 2, 3)]
        peers += [p_x]
        for p in peers:
            pl.semaphore_signal(
                barrier_sem, inc=1, device_id=p,
                device_id_type=pl.DeviceIdType.MESH,
            )
        pl.semaphore_wait(barrier_sem, 7)

        acc_ref[...] = x_ref[...].astype(jnp.bfloat16)

        rdmas = {}

        def own_off(h):
            return base[h] + l1[h] * ROWS

        def a2a_start(h, stage, sub, src_off_fn, dst_acc):
            coord = l1[h] if stage in (0, 3) else l2[h]
            for d in (1, 2, 3):
                pc = (coord + d) % 4
                p = peer_l1(h, pc) if stage in (0, 3) else peer_l2(h, pc)
                soff = src_off_fn(pc) + sub * SR
                src = acc_ref.at[pl.ds(soff, SR), :]
                if dst_acc:
                    dst = acc_ref.at[pl.ds(soff, SR), :]
                else:
                    slot = 3 * (stage == 1) + (d - 1)
                    dst = recv_buf.at[h, slot, pl.ds(sub * SR, SR), :]
                r = pltpu.make_async_remote_copy(
                    src_ref=src,
                    dst_ref=dst,
                    send_sem=send_sems.at[h, stage, d - 1, sub],
                    recv_sem=recv_sems.at[h, stage, d - 1, sub],
                    device_id=p,
                    device_id_type=pl.DeviceIdType.MESH,
                )
                r.start()
                rdmas[(h, stage, d, sub)] = r

        def add_slots(h, first_slot, sub):
            rows = pl.ds(own_off(h) + sub * SR, SR)
            srows = pl.ds(sub * SR, SR)
            acc_ref[rows, :] += (
                recv_buf[h, first_slot, srows, :]
                + recv_buf[h, first_slot + 1, srows, :]
                + recv_buf[h, first_slot + 2, srows, :]
            )

        for h in (0, 1):
            for sub in range(SUBS):
                a2a_start(h, 0, sub, lambda pc, h=h: base[h] + pc * ROWS, False)

        for sub in range(SUBS):
            for h in (0, 1):
                for d in (1, 2, 3):
                    rdmas[(h, 0, d, sub)].wait_recv()
                add_slots(h, 0, sub)
                a2a_start(h, 1, sub, lambda pc, h=h: own_off(h), False)

        for sub in range(SUBS):
            for h in (0, 1):
                for d in (1, 2, 3):
                    rdmas[(h, 1, d, sub)].wait_recv()
                    rdmas[(h, 1, d, sub)].wait_send()
                add_slots(h, 3, sub)
                r = pltpu.make_async_remote_copy(
                    src_ref=acc_ref.at[pl.ds(own_off(h) + sub * SR, SR), :],
                    dst_ref=recv_buf.at[h, 6, pl.ds(sub * SR, SR), :],
                    send_sem=send_sems.at[h, 2, 0, sub],
                    recv_sem=recv_sems.at[h, 2, 0, sub],
                    device_id=p_x,
                    device_id_type=pl.DeviceIdType.MESH,
                )
                r.start()
                rdmas[(h, 2, 1, sub)] = r

        for sub in range(SUBS):
            for h in (0, 1):
                rdmas[(h, 2, 1, sub)].wait_recv()
                rdmas[(h, 2, 1, sub)].wait_send()
                rows = pl.ds(own_off(h) + sub * SR, SR)
                acc_ref[rows, :] += recv_buf[h, 6, pl.ds(sub * SR, SR), :]
                a2a_start(h, 3, sub, lambda pc, h=h: own_off(h), True)

        for sub in range(SUBS):
            for h in (0, 1):
                for d in (1, 2, 3):
                    rdmas[(h, 3, d, sub)].wait_recv()

        out_ref[...] = acc_ref[...].astype(jnp.float32)

        for sub in range(SUBS):
            for h in (0, 1):
                for d in (1, 2, 3):
                    rdmas[(h, 0, d, sub)].wait_send()
                    rdmas[(h, 3, d, sub)].wait_send()

    return pl.pallas_call(
        body,
        out_shape=jax.ShapeDtypeStruct((M, N), jnp.float32),
        in_specs=[pl.BlockSpec(memory_space=pltpu.VMEM)],
        out_specs=pl.BlockSpec(memory_space=pltpu.VMEM),
        scratch_shapes=[
            pltpu.VMEM((M, N), jnp.bfloat16),
            pltpu.VMEM((2, 7, ROWS, N), jnp.bfloat16),
            pltpu.SemaphoreType.DMA((2, 4, 3, SUBS)),
            pltpu.SemaphoreType.DMA((2, 4, 3, SUBS)),
        ],
        compiler_params=pltpu.CompilerParams(collective_id=0),
    )(x2)
